# baseline (speedup 1.0000x reference)
import functools

import jax
import jax.numpy as jnp
from jax import lax
from jax.experimental import pallas as pl
from jax.experimental.pallas import tpu as pltpu
from jax.experimental.pallas import tpu_sc as plsc

N = 10000
E = 160000
NF = 6
H = 128
NG = 16
NC = 21

NPAD = 10240
EPAD = 163840
NTILES = 32
EPT = EPAD // NTILES
CB = 16
NCH_PT = EPT // CB
NCHT = EPAD // CB
ROWS_PER_TILE = NPAD // 16

_SMEAR_COEFF = -0.5 / ((8.0 - 0.0) / (NG - 1)) ** 2
_SMEAR_STEP = 8.0 / (NG - 1)


_RB = 1024
_EB = 4096


def _tc_node1_body(x_ref, wne_ref, bne_ref, wfa_ref, wsa_ref, wfb_ref, wsb_ref,
                   h_ref, t_ref):
    h = jnp.dot(x_ref[...], wne_ref[...], preferred_element_type=jnp.float32)
    h = h + bne_ref[...]
    h_ref[...] = h
    t_ref[0, :, :H] = jnp.dot(h, wfa_ref[...], preferred_element_type=jnp.float32)
    t_ref[0, :, H:] = jnp.dot(h, wsa_ref[...], preferred_element_type=jnp.float32)
    t_ref[1, :, :H] = jnp.dot(h, wfb_ref[...], preferred_element_type=jnp.float32)
    t_ref[1, :, H:] = jnp.dot(h, wsb_ref[...], preferred_element_type=jnp.float32)


def _tc_edge_body(ea_ref, wf1c_ref, bf1_ref, ws1c_ref, bs1_ref,
                  wf2c_ref, bf2_ref, ws2c_ref, bs2_ref, c1_ref, c2_ref):
    d = ea_ref[...]
    off = lax.broadcasted_iota(jnp.int32, (1, NG), 1).astype(jnp.float32) * _SMEAR_STEP
    ea = jnp.exp(_SMEAR_COEFF * (d - off) ** 2)
    c1_ref[:, :H] = jnp.dot(ea, wf1c_ref[...], preferred_element_type=jnp.float32) + bf1_ref[...]
    c1_ref[:, H:] = jnp.dot(ea, ws1c_ref[...], preferred_element_type=jnp.float32) + bs1_ref[...]
    c2_ref[:, :H] = jnp.dot(ea, wf2c_ref[...], preferred_element_type=jnp.float32) + bf2_ref[...]
    c2_ref[:, H:] = jnp.dot(ea, ws2c_ref[...], preferred_element_type=jnp.float32) + bs2_ref[...]


def _tc_node2_body(p0_ref, p1_ref, hprev_ref, g_ref, b_ref, m_ref, v_ref,
                   wfa_ref, wsa_ref, wfb_ref, wsb_ref, h_ref, t_ref):
    agg = p0_ref[...] + p1_ref[...]
    scale = g_ref[...] * lax.rsqrt(v_ref[...] + 1e-5)
    h = jnp.maximum((agg - m_ref[...]) * scale + b_ref[...] + hprev_ref[...], 0.0)
    h_ref[...] = h
    t_ref[0, :, :H] = jnp.dot(h, wfa_ref[...], preferred_element_type=jnp.float32)
    t_ref[0, :, H:] = jnp.dot(h, wsa_ref[...], preferred_element_type=jnp.float32)
    t_ref[1, :, :H] = jnp.dot(h, wfb_ref[...], preferred_element_type=jnp.float32)
    t_ref[1, :, H:] = jnp.dot(h, wsb_ref[...], preferred_element_type=jnp.float32)


def _tc_final_body(p0_ref, p1_ref, hprev_ref, g_ref, b_ref, m_ref, v_ref,
                   wfc_ref, bfc_ref, out_ref):
    agg = p0_ref[...] + p1_ref[...]
    scale = g_ref[...] * lax.rsqrt(v_ref[...] + 1e-5)
    h = jnp.maximum((agg - m_ref[...]) * scale + b_ref[...] + hprev_ref[...], 0.0)
    out_ref[...] = jnp.dot(h, wfc_ref[...], preferred_element_type=jnp.float32) + bfc_ref[...]


def _row_spec(block_rows, cols):
    return pl.BlockSpec((block_rows, cols), lambda i: (i, 0))


def _full_spec(shape):
    return pl.BlockSpec(shape, lambda i: (0,) * len(shape))


_sc_mesh = plsc.VectorSubcoreMesh(core_axis_name="c", subcore_axis_name="s")


@functools.partial(
    pl.kernel,
    out_type=jax.ShapeDtypeStruct((2 * NPAD, H), jnp.float32),
    mesh=_sc_mesh,
    scratch_types=[
        pltpu.VMEM((2, CB), jnp.int32),
        pltpu.VMEM((2, CB), jnp.int32),
        pltpu.VMEM((2, CB), jnp.int32),
        pltpu.VMEM((2, CB), jnp.int32),
        pltpu.VMEM((2, CB, 2 * H), jnp.float32),
        pltpu.VMEM((2, CB, 2 * H), jnp.float32),
        pltpu.VMEM((CB, 2 * H), jnp.float32),
        pltpu.VMEM((CB, 2 * H), jnp.float32),
        pltpu.VMEM((CB, H), jnp.float32),
        pltpu.VMEM((CB, H), jnp.float32),
        pltpu.VMEM_SHARED((NPAD, H), jnp.float32),
        pltpu.SemaphoreType.DMA,
        pltpu.SemaphoreType.DMA,
        pltpu.SemaphoreType.DMA,
        pltpu.SemaphoreType.DMA,
        pltpu.SemaphoreType.DMA,
        pltpu.SemaphoreType.DMA,
        pltpu.SemaphoreType.DMA,
        pltpu.SemaphoreType.DMA,
        pltpu.SemaphoreType.DMA,
        pltpu.SemaphoreType.DMA,
    ],
)
def _sc_edge(t_hbm, c_hbm, gi_hbm, zero_hbm, out_hbm,
             gi0, gi1, gi2, gi3, rw0, rw1, cr0, cr1, ms0, ms1, acc,
             si0, si1, si2, si3, sg0, sg1, sc0, sc1, ss0, ss1):
    cid = lax.axis_index("c")
    sid = lax.axis_index("s")
    wid = cid * 16 + sid
    GI = (gi0, gi1, gi2, gi3)
    SI = (si0, si1, si2, si3)
    RW = (rw0, rw1)
    CR = (cr0, cr1)
    MS = (ms0, ms1)
    SG = (sg0, sg1)
    SC = (sc0, sc1)
    SS = (ss0, ss1)

    r0 = sid * ROWS_PER_TILE
    pltpu.sync_copy(zero_hbm.at[pl.ds(r0, ROWS_PER_TILE)],
                    acc.at[pl.ds(r0, ROWS_PER_TILE)])
    plsc.subcore_barrier()

    ch0 = wid * NCH_PT

    def issue_idx(k, q):
        pltpu.async_copy(gi_hbm.at[ch0 + k], GI[q], SI[q])

    def wait_idx(q):
        pltpu.make_async_copy(gi_hbm.at[0], GI[q], SI[q]).wait()

    def issue_data(k, b, q):
        pltpu.async_copy(t_hbm.at[GI[q].at[0]], RW[b].at[0], SG[b])
        pltpu.async_copy(t_hbm.at[GI[q].at[1]], RW[b].at[1], SG[b])
        pltpu.async_copy(c_hbm.at[pl.ds((ch0 + k) * CB, CB)], CR[b], SC[b])

    def wait_data(b, q):
        pltpu.make_async_copy(t_hbm.at[GI[q].at[0]], RW[b].at[0], SG[b]).wait()
        pltpu.make_async_copy(t_hbm.at[GI[q].at[1]], RW[b].at[1], SG[b]).wait()
        pltpu.make_async_copy(c_hbm.at[pl.ds(0, CB)], CR[b], SC[b]).wait()

    def issue_scatter(b, q):
        pltpu.async_copy(MS[b], acc.at[GI[q].at[0]], SS[b], add=True)

    def wait_scatter(b, q):
        pltpu.make_async_copy(MS[b], acc.at[GI[q].at[0]], SS[b]).wait()

    pltpu.sync_copy(gi_hbm.at[ch0], gi0)
    issue_data(0, 0, 0)
    issue_idx(1, 1)

    def _steps(k4, carry):
        for j in range(4):
            b = j % 2
            bo = 1 - b
            q = j
            k = k4 * 4 + j

            @pl.when(k + 1 < NCH_PT)
            def _(bo=bo, q=q, k=k):
                wait_idx((q + 1) % 4)
                issue_data(k + 1, bo, (q + 1) % 4)

            wait_data(b, q)

            @pl.when(k >= 2)
            def _(b=b, q=q):
                wait_scatter(b, (q + 2) % 4)

            @pl.when(k + 2 < NCH_PT)
            def _(q=q, k=k):
                issue_idx(k + 2, (q + 2) % 4)

            @plsc.parallel_loop(0, CB, unroll=2)
            def _row(r, b=b):
                for jj in range(H // 16):
                    o = jj * 16
                    uf = (RW[b][0, r, pl.ds(o, 16)] + RW[b][1, r, pl.ds(o, 16)]
                          + CR[b][r, pl.ds(o, 16)])
                    us = (RW[b][0, r, pl.ds(H + o, 16)] + RW[b][1, r, pl.ds(H + o, 16)]
                          + CR[b][r, pl.ds(H + o, 16)])
                    sig = 1.0 / (1.0 + jnp.exp(-uf))
                    t = jnp.exp(-jnp.abs(us))
                    lp = t * (0.99997024 + t * (-0.49933395 + t * (0.32751171
                         + t * (-0.22396690 + t * (0.13198966 + t * (-0.05326748
                         + t * 0.01024383))))))
                    sp = jnp.maximum(us, 0.0) + lp
                    MS[b][r, pl.ds(o, 16)] = sig * sp

            issue_scatter(b, q)
        return carry

    lax.fori_loop(0, NCH_PT // 4, _steps, 0)

    wait_scatter(0, 2)
    wait_scatter(1, 3)
    plsc.subcore_barrier()

    pltpu.sync_copy(acc.at[pl.ds(r0, ROWS_PER_TILE)],
                    out_hbm.at[pl.ds(cid * NPAD + r0, ROWS_PER_TILE)])


def kernel(x, edge_index, edge_attr, W_ne, b_ne,
           Wf1, bf1, Ws1, bs1, bn1_g, bn1_b, bn1_m, bn1_v,
           Wf2, bf2, Ws2, bs2, bn2_g, bn2_b, bn2_m, bn2_v,
           W_fc, b_fc):
    f32 = jnp.float32
    xpad = jnp.pad(x, ((0, NPAD - N), (0, 8 - NF)))
    wne = jnp.pad(W_ne, ((0, 8 - NF), (0, 0)))
    pad_idx = jnp.full((EPAD - E,), N, jnp.int32)
    src_idx = jnp.concatenate([edge_index[0], pad_idx])
    dst_idx = jnp.concatenate([edge_index[1], pad_idx])
    gidx = jnp.stack([dst_idx.reshape(NCHT, CB),
                      src_idx.reshape(NCHT, CB) + NPAD], axis=1)
    ea_col = jnp.pad(edge_attr, (0, EPAD - E)).reshape(EPAD, 1)
    wfc_pad = jnp.pad(W_fc, ((0, 0), (0, 32 - NC)))
    bfc_pad = jnp.pad(b_fc, (0, 32 - NC)).reshape(1, 32)
    zeros_nh = jnp.zeros((NPAD, H), f32)

    def row1(a):
        return a.reshape(1, H)

    n_grid = NPAD // _RB
    e_grid = EPAD // _EB

    h0, t1 = pl.pallas_call(
        _tc_node1_body,
        grid=(n_grid,),
        in_specs=[
            _row_spec(_RB, 8),
            _full_spec((8, H)), _full_spec((1, H)),
            _full_spec((H, H)), _full_spec((H, H)),
            _full_spec((H, H)), _full_spec((H, H)),
        ],
        out_specs=[
            _row_spec(_RB, H),
            pl.BlockSpec((2, _RB, 2 * H), lambda i: (0, i, 0)),
        ],
        out_shape=[
            jax.ShapeDtypeStruct((NPAD, H), f32),
            jax.ShapeDtypeStruct((2, NPAD, 2 * H), f32),
        ],
    )(xpad, wne, b_ne.reshape(1, H),
      Wf1[:H], Ws1[:H], Wf1[H:2 * H], Ws1[H:2 * H])

    c1, c2 = pl.pallas_call(
        _tc_edge_body,
        grid=(e_grid,),
        in_specs=[
            _row_spec(_EB, 1),
            _full_spec((NG, H)), _full_spec((1, H)),
            _full_spec((NG, H)), _full_spec((1, H)),
            _full_spec((NG, H)), _full_spec((1, H)),
            _full_spec((NG, H)), _full_spec((1, H)),
        ],
        out_specs=[_row_spec(_EB, 2 * H), _row_spec(_EB, 2 * H)],
        out_shape=[
            jax.ShapeDtypeStruct((EPAD, 2 * H), f32),
            jax.ShapeDtypeStruct((EPAD, 2 * H), f32),
        ],
    )(ea_col, Wf1[2 * H:], row1(bf1), Ws1[2 * H:], row1(bs1),
      Wf2[2 * H:], row1(bf2), Ws2[2 * H:], row1(bs2))

    parts1 = _sc_edge(t1.reshape(2 * NPAD, 2 * H), c1, gidx, zeros_nh)
    p1a = parts1[:NPAD]
    p1b = parts1[NPAD:]

    h1, t2 = pl.pallas_call(
        _tc_node2_body,
        grid=(n_grid,),
        in_specs=[
            _row_spec(_RB, H), _row_spec(_RB, H), _row_spec(_RB, H),
            _full_spec((1, H)), _full_spec((1, H)),
            _full_spec((1, H)), _full_spec((1, H)),
            _full_spec((H, H)), _full_spec((H, H)),
            _full_spec((H, H)), _full_spec((H, H)),
        ],
        out_specs=[
            _row_spec(_RB, H),
            pl.BlockSpec((2, _RB, 2 * H), lambda i: (0, i, 0)),
        ],
        out_shape=[
            jax.ShapeDtypeStruct((NPAD, H), f32),
            jax.ShapeDtypeStruct((2, NPAD, 2 * H), f32),
        ],
    )(p1a, p1b, h0, row1(bn1_g), row1(bn1_b), row1(bn1_m), row1(bn1_v),
      Wf2[:H], Ws2[:H], Wf2[H:2 * H], Ws2[H:2 * H])

    parts2 = _sc_edge(t2.reshape(2 * NPAD, 2 * H), c2, gidx, zeros_nh)
    p2a = parts2[:NPAD]
    p2b = parts2[NPAD:]

    logits = pl.pallas_call(
        _tc_final_body,
        grid=(n_grid,),
        in_specs=[
            _row_spec(_RB, H), _row_spec(_RB, H), _row_spec(_RB, H),
            _full_spec((1, H)), _full_spec((1, H)),
            _full_spec((1, H)), _full_spec((1, H)),
            _full_spec((H, 32)), _full_spec((1, 32)),
        ],
        out_specs=_row_spec(_RB, 32),
        out_shape=jax.ShapeDtypeStruct((NPAD, 32), f32),
    )(p2a, p2b, h1, row1(bn2_g), row1(bn2_b), row1(bn2_m), row1(bn2_v),
      wfc_pad, bfc_pad)

    return logits[:N, :NC]


# speedup vs baseline: 1.2770x; 1.2770x over previous
import functools

import jax
import jax.numpy as jnp
from jax import lax
from jax.experimental import pallas as pl
from jax.experimental.pallas import tpu as pltpu
from jax.experimental.pallas import tpu_sc as plsc

N = 10000
E = 160000
NF = 6
H = 128
NG = 16
NC = 21

NPAD = 10240
EPAD = 163840
NTILES = 32
EPT = EPAD // NTILES
CB = 32
NCH_PT = EPT // CB
NCHT = EPAD // CB
ROWS_PER_TILE = NPAD // 16

_SMEAR_COEFF = -0.5 / ((8.0 - 0.0) / (NG - 1)) ** 2
_SMEAR_STEP = 8.0 / (NG - 1)
_HI_MASK = -65536


_RB = 1024
_EB = 4096


def _pack_pair(f, s):
    fb = jax.lax.bitcast_convert_type(f, jnp.int32)
    sb = jax.lax.bitcast_convert_type(s, jnp.int32)
    fr = fb + 0x7FFF + jax.lax.shift_right_logical(fb, 16).astype(jnp.int32) % 2
    sr = sb + 0x7FFF + jax.lax.shift_right_logical(sb, 16).astype(jnp.int32) % 2
    lo = jax.lax.shift_right_logical(fr, 16)
    hi = sr & _HI_MASK
    return lo | hi


def _tc_node1_body(x_ref, wne_ref, bne_ref, wfa_ref, wsa_ref, wfb_ref, wsb_ref,
                   h_ref, t_ref):
    h = jnp.dot(x_ref[...], wne_ref[...], preferred_element_type=jnp.float32)
    h = h + bne_ref[...]
    h_ref[...] = h
    t_ref[0] = _pack_pair(
        jnp.dot(h, wfa_ref[...], preferred_element_type=jnp.float32),
        jnp.dot(h, wsa_ref[...], preferred_element_type=jnp.float32))
    t_ref[1] = _pack_pair(
        jnp.dot(h, wfb_ref[...], preferred_element_type=jnp.float32),
        jnp.dot(h, wsb_ref[...], preferred_element_type=jnp.float32))


def _tc_edge_body(ea_ref, wf1c_ref, bf1_ref, ws1c_ref, bs1_ref,
                  wf2c_ref, bf2_ref, ws2c_ref, bs2_ref, c1_ref, c2_ref):
    d = ea_ref[...]
    off = lax.broadcasted_iota(jnp.int32, (1, NG), 1).astype(jnp.float32) * _SMEAR_STEP
    ea = jnp.exp(_SMEAR_COEFF * (d - off) ** 2)
    c1_ref[...] = _pack_pair(
        jnp.dot(ea, wf1c_ref[...], preferred_element_type=jnp.float32) + bf1_ref[...],
        jnp.dot(ea, ws1c_ref[...], preferred_element_type=jnp.float32) + bs1_ref[...])
    c2_ref[...] = _pack_pair(
        jnp.dot(ea, wf2c_ref[...], preferred_element_type=jnp.float32) + bf2_ref[...],
        jnp.dot(ea, ws2c_ref[...], preferred_element_type=jnp.float32) + bs2_ref[...])


def _tc_node2_body(p0_ref, p1_ref, hprev_ref, g_ref, b_ref, m_ref, v_ref,
                   wfa_ref, wsa_ref, wfb_ref, wsb_ref, h_ref, t_ref):
    agg = p0_ref[...] + p1_ref[...]
    scale = g_ref[...] * lax.rsqrt(v_ref[...] + 1e-5)
    h = jnp.maximum((agg - m_ref[...]) * scale + b_ref[...] + hprev_ref[...], 0.0)
    h_ref[...] = h
    t_ref[0] = _pack_pair(
        jnp.dot(h, wfa_ref[...], preferred_element_type=jnp.float32),
        jnp.dot(h, wsa_ref[...], preferred_element_type=jnp.float32))
    t_ref[1] = _pack_pair(
        jnp.dot(h, wfb_ref[...], preferred_element_type=jnp.float32),
        jnp.dot(h, wsb_ref[...], preferred_element_type=jnp.float32))


def _tc_final_body(p0_ref, p1_ref, hprev_ref, g_ref, b_ref, m_ref, v_ref,
                   wfc_ref, bfc_ref, out_ref):
    agg = p0_ref[...] + p1_ref[...]
    scale = g_ref[...] * lax.rsqrt(v_ref[...] + 1e-5)
    h = jnp.maximum((agg - m_ref[...]) * scale + b_ref[...] + hprev_ref[...], 0.0)
    out_ref[...] = jnp.dot(h, wfc_ref[...], preferred_element_type=jnp.float32) + bfc_ref[...]


def _row_spec(block_rows, cols):
    return pl.BlockSpec((block_rows, cols), lambda i: (i, 0))


def _full_spec(shape):
    return pl.BlockSpec(shape, lambda i: (0,) * len(shape))


_sc_mesh = plsc.VectorSubcoreMesh(core_axis_name="c", subcore_axis_name="s")


@functools.partial(
    pl.kernel,
    out_type=jax.ShapeDtypeStruct((2 * NPAD, H), jnp.float32),
    mesh=_sc_mesh,
    scratch_types=[
        pltpu.VMEM((2, CB), jnp.int32),
        pltpu.VMEM((2, CB), jnp.int32),
        pltpu.VMEM((2, CB), jnp.int32),
        pltpu.VMEM((2, CB), jnp.int32),
        pltpu.VMEM((2, CB, H), jnp.int32),
        pltpu.VMEM((2, CB, H), jnp.int32),
        pltpu.VMEM((CB, H), jnp.int32),
        pltpu.VMEM((CB, H), jnp.int32),
        pltpu.VMEM((CB, H), jnp.float32),
        pltpu.VMEM((CB, H), jnp.float32),
        pltpu.VMEM_SHARED((NPAD, H), jnp.float32),
        pltpu.SemaphoreType.DMA,
        pltpu.SemaphoreType.DMA,
        pltpu.SemaphoreType.DMA,
        pltpu.SemaphoreType.DMA,
        pltpu.SemaphoreType.DMA,
        pltpu.SemaphoreType.DMA,
        pltpu.SemaphoreType.DMA,
        pltpu.SemaphoreType.DMA,
        pltpu.SemaphoreType.DMA,
        pltpu.SemaphoreType.DMA,
    ],
)
def _sc_edge(t_hbm, c_hbm, gi_hbm, zero_hbm, out_hbm,
             gi0, gi1, gi2, gi3, rw0, rw1, cr0, cr1, ms0, ms1, acc,
             si0, si1, si2, si3, sg0, sg1, sc0, sc1, ss0, ss1):
    cid = lax.axis_index("c")
    sid = lax.axis_index("s")
    wid = cid * 16 + sid
    GI = (gi0, gi1, gi2, gi3)
    SI = (si0, si1, si2, si3)
    RW = (rw0, rw1)
    CR = (cr0, cr1)
    MS = (ms0, ms1)
    SG = (sg0, sg1)
    SC = (sc0, sc1)
    SS = (ss0, ss1)

    r0 = sid * ROWS_PER_TILE
    pltpu.sync_copy(zero_hbm.at[pl.ds(r0, ROWS_PER_TILE)],
                    acc.at[pl.ds(r0, ROWS_PER_TILE)])
    plsc.subcore_barrier()

    ch0 = wid * NCH_PT

    def issue_idx(k, q):
        pltpu.async_copy(gi_hbm.at[ch0 + k], GI[q], SI[q])

    def wait_idx(q):
        pltpu.make_async_copy(gi_hbm.at[0], GI[q], SI[q]).wait()

    def issue_data(k, b, q):
        pltpu.async_copy(t_hbm.at[GI[q].at[0]], RW[b].at[0], SG[b])
        pltpu.async_copy(t_hbm.at[GI[q].at[1]], RW[b].at[1], SG[b])
        pltpu.async_copy(c_hbm.at[pl.ds((ch0 + k) * CB, CB)], CR[b], SC[b])

    def wait_data(b, q):
        pltpu.make_async_copy(t_hbm.at[GI[q].at[0]], RW[b].at[0], SG[b]).wait()
        pltpu.make_async_copy(t_hbm.at[GI[q].at[1]], RW[b].at[1], SG[b]).wait()
        pltpu.make_async_copy(c_hbm.at[pl.ds(0, CB)], CR[b], SC[b]).wait()

    def issue_scatter(b, q):
        pltpu.async_copy(MS[b], acc.at[GI[q].at[0]], SS[b], add=True)

    def wait_scatter(b, q):
        pltpu.make_async_copy(MS[b], acc.at[GI[q].at[0]], SS[b]).wait()

    pltpu.sync_copy(gi_hbm.at[ch0], gi0)
    issue_data(0, 0, 0)
    issue_idx(1, 1)

    def _lo(w):
        return jax.lax.bitcast_convert_type(jax.lax.shift_left(w, 16), jnp.float32)

    def _hi(w):
        return jax.lax.bitcast_convert_type(w & _HI_MASK, jnp.float32)

    def _steps(k4, carry):
        for j in range(4):
            b = j % 2
            bo = 1 - b
            q = j
            k = k4 * 4 + j

            @pl.when(k + 1 < NCH_PT)
            def _(bo=bo, q=q, k=k):
                wait_idx((q + 1) % 4)
                issue_data(k + 1, bo, (q + 1) % 4)

            wait_data(b, q)

            @pl.when(k >= 2)
            def _(b=b, q=q):
                wait_scatter(b, (q + 2) % 4)

            @pl.when(k + 2 < NCH_PT)
            def _(q=q, k=k):
                issue_idx(k + 2, (q + 2) % 4)

            @plsc.parallel_loop(0, CB, unroll=2)
            def _row(r, b=b):
                for g in range(H // 16):
                    o = g * 16
                    dw = RW[b][0, r, pl.ds(o, 16)]
                    sw = RW[b][1, r, pl.ds(o, 16)]
                    cw = CR[b][r, pl.ds(o, 16)]
                    uf = _lo(dw) + _lo(sw) + _lo(cw)
                    us = _hi(dw) + _hi(sw) + _hi(cw)
                    sig = 1.0 / (1.0 + jnp.exp(-uf))
                    t = jnp.exp(-jnp.abs(us))
                    sq = t / (t + 2.0)
                    qq = sq * sq
                    lp = 2.0 * sq * (1.0 + qq * (1.0 / 3.0 + qq * (0.2 + qq * (1.0 / 7.0))))
                    sp = jnp.maximum(us, 0.0) + lp
                    MS[b][r, pl.ds(o, 16)] = sig * sp

            issue_scatter(b, q)
        return carry

    lax.fori_loop(0, NCH_PT // 4, _steps, 0)

    wait_scatter(0, 2)
    wait_scatter(1, 3)
    plsc.subcore_barrier()

    pltpu.sync_copy(acc.at[pl.ds(r0, ROWS_PER_TILE)],
                    out_hbm.at[pl.ds(cid * NPAD + r0, ROWS_PER_TILE)])


def kernel(x, edge_index, edge_attr, W_ne, b_ne,
           Wf1, bf1, Ws1, bs1, bn1_g, bn1_b, bn1_m, bn1_v,
           Wf2, bf2, Ws2, bs2, bn2_g, bn2_b, bn2_m, bn2_v,
           W_fc, b_fc):
    f32 = jnp.float32
    xpad = jnp.pad(x, ((0, NPAD - N), (0, 8 - NF)))
    wne = jnp.pad(W_ne, ((0, 8 - NF), (0, 0)))
    pad_idx = jnp.full((EPAD - E,), N, jnp.int32)
    src_idx = jnp.concatenate([edge_index[0], pad_idx])
    dst_idx = jnp.concatenate([edge_index[1], pad_idx])
    gidx = jnp.stack([dst_idx.reshape(NCHT, CB),
                      src_idx.reshape(NCHT, CB) + NPAD], axis=1)
    ea_col = jnp.pad(edge_attr, (0, EPAD - E)).reshape(EPAD, 1)
    wfc_pad = jnp.pad(W_fc, ((0, 0), (0, 32 - NC)))
    bfc_pad = jnp.pad(b_fc, (0, 32 - NC)).reshape(1, 32)
    zeros_nh = jnp.zeros((NPAD, H), f32)

    def row1(a):
        return a.reshape(1, H)

    n_grid = NPAD // _RB
    e_grid = EPAD // _EB

    h0, t1 = pl.pallas_call(
        _tc_node1_body,
        grid=(n_grid,),
        in_specs=[
            _row_spec(_RB, 8),
            _full_spec((8, H)), _full_spec((1, H)),
            _full_spec((H, H)), _full_spec((H, H)),
            _full_spec((H, H)), _full_spec((H, H)),
        ],
        out_specs=[
            _row_spec(_RB, H),
            pl.BlockSpec((2, _RB, H), lambda i: (0, i, 0)),
        ],
        out_shape=[
            jax.ShapeDtypeStruct((NPAD, H), f32),
            jax.ShapeDtypeStruct((2, NPAD, H), jnp.int32),
        ],
    )(xpad, wne, b_ne.reshape(1, H),
      Wf1[:H], Ws1[:H], Wf1[H:2 * H], Ws1[H:2 * H])

    c1, c2 = pl.pallas_call(
        _tc_edge_body,
        grid=(e_grid,),
        in_specs=[
            _row_spec(_EB, 1),
            _full_spec((NG, H)), _full_spec((1, H)),
            _full_spec((NG, H)), _full_spec((1, H)),
            _full_spec((NG, H)), _full_spec((1, H)),
            _full_spec((NG, H)), _full_spec((1, H)),
        ],
        out_specs=[_row_spec(_EB, H), _row_spec(_EB, H)],
        out_shape=[
            jax.ShapeDtypeStruct((EPAD, H), jnp.int32),
            jax.ShapeDtypeStruct((EPAD, H), jnp.int32),
        ],
    )(ea_col, Wf1[2 * H:], row1(bf1), Ws1[2 * H:], row1(bs1),
      Wf2[2 * H:], row1(bf2), Ws2[2 * H:], row1(bs2))

    parts1 = _sc_edge(t1.reshape(2 * NPAD, H), c1, gidx, zeros_nh)
    p1a = parts1[:NPAD]
    p1b = parts1[NPAD:]

    h1, t2 = pl.pallas_call(
        _tc_node2_body,
        grid=(n_grid,),
        in_specs=[
            _row_spec(_RB, H), _row_spec(_RB, H), _row_spec(_RB, H),
            _full_spec((1, H)), _full_spec((1, H)),
            _full_spec((1, H)), _full_spec((1, H)),
            _full_spec((H, H)), _full_spec((H, H)),
            _full_spec((H, H)), _full_spec((H, H)),
        ],
        out_specs=[
            _row_spec(_RB, H),
            pl.BlockSpec((2, _RB, H), lambda i: (0, i, 0)),
        ],
        out_shape=[
            jax.ShapeDtypeStruct((NPAD, H), f32),
            jax.ShapeDtypeStruct((2, NPAD, H), jnp.int32),
        ],
    )(p1a, p1b, h0, row1(bn1_g), row1(bn1_b), row1(bn1_m), row1(bn1_v),
      Wf2[:H], Ws2[:H], Wf2[H:2 * H], Ws2[H:2 * H])

    parts2 = _sc_edge(t2.reshape(2 * NPAD, H), c2, gidx, zeros_nh)
    p2a = parts2[:NPAD]
    p2b = parts2[NPAD:]

    logits = pl.pallas_call(
        _tc_final_body,
        grid=(n_grid,),
        in_specs=[
            _row_spec(_RB, H), _row_spec(_RB, H), _row_spec(_RB, H),
            _full_spec((1, H)), _full_spec((1, H)),
            _full_spec((1, H)), _full_spec((1, H)),
            _full_spec((H, 32)), _full_spec((1, 32)),
        ],
        out_specs=_row_spec(_RB, 32),
        out_shape=jax.ShapeDtypeStruct((NPAD, 32), f32),
    )(p2a, p2b, h1, row1(bn2_g), row1(bn2_b), row1(bn2_m), row1(bn2_v),
      wfc_pad, bfc_pad)

    return logits[:N, :NC]
